# baseline (speedup 1.0000x reference)
import jax
import jax.numpy as jnp
import numpy as np
from jax import lax
from jax.experimental import pallas as pl
from jax.experimental.pallas import tpu as pltpu
from jax.experimental.pallas import tpu_sc as plsc

N_NODES = 10000
IN_CH = 128
OUT_CH = 16
HEADS = 8
FEAT = HEADS * OUT_CH
ZW = 160
PW = 80
AW = 144

NC = 2
NS = 16
NW = NC * NS

EDGE_K = 50
CHUNKS = 200
PHASE = 50
DRAIN = (50,) * 12 + (25,)

_JREP8 = np.repeat(np.eye(HEADS, dtype=np.float32), OUT_CH, axis=0)
_JFULL = np.zeros((FEAT, ZW), np.float32)
_JFULL[:, :FEAT] = np.repeat(_JREP8, OUT_CH, axis=1)
_JFULL[:, FEAT:FEAT + HEADS] = _JREP8
_RMAT = np.repeat(np.eye(HEADS, dtype=np.float32), OUT_CH, axis=1)


def _proj_body(x_ref, w_ref, af_ref, j_ref, z_ref):
    xp = jnp.dot(x_ref[...], w_ref[...], preferred_element_type=jnp.float32)
    m = xp * af_ref[...]
    e = jnp.exp(jnp.dot(m, j_ref[...], preferred_element_type=jnp.float32))
    b = jnp.concatenate(
        [xp, jnp.ones((xp.shape[0], ZW - FEAT), jnp.float32)], axis=1)
    z = b * e
    lo = z[:, :PW].astype(jnp.bfloat16).astype(jnp.float32)
    hi = z[:, PW:].astype(jnp.bfloat16).astype(jnp.float32)
    lo_b = jax.lax.bitcast_convert_type(lo, jnp.int32)
    hi_b = jax.lax.bitcast_convert_type(hi, jnp.int32)
    z_ref[...] = jax.lax.shift_right_logical(lo_b, 16) | (
        hi_b & jnp.int32(-65536))


def _combine_body(p_ref, r_ref, o_ref):
    t = p_ref[0] + p_ref[1]
    u = t[:, :FEAT]
    s = t[:, FEAT:FEAT + HEADS]
    srep = jnp.dot(s, r_ref[...], preferred_element_type=jnp.float32)
    o_ref[...] = u / (srep + 1e-16)


def _edge_body(z_hbm, row_hbm, col_hbm, out_hbm, accum_ref):
    cid = lax.axis_index("c")
    sid = lax.axis_index("s")
    wid = cid * NS + sid

    def scoped(row_v, col_v, bb_a, bb_b, fbuf_a, fbuf_b, gs_a, gs_b, ss_a, ss_b):
        fbuf = fbuf_a
        def zrow(i, _):
            for o in range(AW // 16):
                fbuf[i, pl.ds(o * 16, 16)] = jnp.zeros((16,), jnp.float32)
            return 0
        lax.fori_loop(0, EDGE_K, zrow, 0)
        rows_per_tile = N_NODES // NS
        r0 = sid * rows_per_tile
        for n in DRAIN:
            pltpu.sync_copy(fbuf.at[pl.ds(0, n)], accum_ref.at[pl.ds(r0, n)])
            r0 += n
        plsc.subcore_barrier()

        def convert(bb, fb):
            def crow(r2, _):
                for u in range(2):
                    r = 2 * r2 + u
                    for g in range(5):
                        w = bb[r, pl.ds(g * 16, 16)]
                        lo = jax.lax.bitcast_convert_type(
                            jnp.left_shift(w, 16), jnp.float32)
                        fb[r, pl.ds(g * 16, 16)] = lo
                        if g < 4:
                            hi = jax.lax.bitcast_convert_type(
                                jnp.bitwise_and(w, jnp.int32(-65536)), jnp.float32)
                            fb[r, pl.ds(PW + g * 16, 16)] = hi
                return 0
            lax.fori_loop(0, EDGE_K // 2, crow, 0)

        def phase(ph, _):
            base = wid * CHUNKS + ph * PHASE
            pltpu.sync_copy(row_hbm.at[pl.ds(base, PHASE)], row_v)
            pltpu.sync_copy(col_hbm.at[pl.ds(base, PHASE)], col_v)
            pltpu.async_copy(z_hbm.at[row_v.at[0]], bb_a, gs_a)

            def group(g, _):
                j0 = 2 * g
                pltpu.make_async_copy(z_hbm.at[row_v.at[j0]], bb_a, gs_a).wait()
                pltpu.async_copy(z_hbm.at[row_v.at[j0 + 1]], bb_b, gs_b)

                @pl.when(g > 0)
                def _():
                    pltpu.make_async_copy(
                        fbuf_a, accum_ref.at[col_v.at[j0 - 2]], ss_a).wait()

                convert(bb_a, fbuf_a)
                pltpu.async_copy(fbuf_a, accum_ref.at[col_v.at[j0]], ss_a, add=True)
                pltpu.make_async_copy(z_hbm.at[row_v.at[j0 + 1]], bb_b, gs_b).wait()

                @pl.when(g < PHASE // 2 - 1)
                def _():
                    pltpu.async_copy(z_hbm.at[row_v.at[j0 + 2]], bb_a, gs_a)

                @pl.when(g > 0)
                def _():
                    pltpu.make_async_copy(
                        fbuf_b, accum_ref.at[col_v.at[j0 - 1]], ss_b).wait()

                convert(bb_b, fbuf_b)
                pltpu.async_copy(fbuf_b, accum_ref.at[col_v.at[j0 + 1]], ss_b,
                                 add=True)
                return 0
            lax.fori_loop(0, PHASE // 2, group, 0)
            pltpu.make_async_copy(
                fbuf_a, accum_ref.at[col_v.at[PHASE - 2]], ss_a).wait()
            pltpu.make_async_copy(
                fbuf_b, accum_ref.at[col_v.at[PHASE - 1]], ss_b).wait()
            return 0
        lax.fori_loop(0, CHUNKS // PHASE, phase, 0)
        plsc.subcore_barrier()

        r0 = sid * rows_per_tile
        for n in DRAIN:
            pltpu.sync_copy(accum_ref.at[pl.ds(r0, n)], fbuf.at[pl.ds(0, n)])
            pltpu.sync_copy(fbuf.at[pl.ds(0, n)], out_hbm.at[cid, pl.ds(r0, n)])
            r0 += n

    pl.run_scoped(
        scoped,
        pltpu.VMEM((PHASE, EDGE_K), jnp.int32),
        pltpu.VMEM((PHASE, EDGE_K), jnp.int32),
        pltpu.VMEM((EDGE_K, PW), jnp.int32),
        pltpu.VMEM((EDGE_K, PW), jnp.int32),
        pltpu.VMEM((EDGE_K, AW), jnp.float32),
        pltpu.VMEM((EDGE_K, AW), jnp.float32),
        pltpu.SemaphoreType.DMA,
        pltpu.SemaphoreType.DMA,
        pltpu.SemaphoreType.DMA,
        pltpu.SemaphoreType.DMA,
    )


def kernel(x, edge_index, W, a_src, a_dst):
    row = edge_index[0].astype(jnp.int32).reshape(-1, EDGE_K)
    col = edge_index[1].astype(jnp.int32).reshape(-1, EDGE_K)
    a_flat = a_src.reshape(1, FEAT)

    blk = 1000
    grid = N_NODES // blk
    z = pl.pallas_call(
        _proj_body,
        grid=(grid,),
        in_specs=[
            pl.BlockSpec((blk, IN_CH), lambda i: (i, 0)),
            pl.BlockSpec((IN_CH, FEAT), lambda i: (0, 0)),
            pl.BlockSpec((1, FEAT), lambda i: (0, 0)),
            pl.BlockSpec((IN_CH, ZW), lambda i: (0, 0)),
        ],
        out_specs=pl.BlockSpec((blk, PW), lambda i: (i, 0)),
        out_shape=jax.ShapeDtypeStruct((N_NODES, PW), jnp.int32),
    )(x, W, a_flat, jnp.asarray(_JFULL))

    mesh = plsc.VectorSubcoreMesh(
        core_axis_name="c", subcore_axis_name="s", num_cores=NC, num_subcores=NS)
    edge_k = pl.kernel(
        _edge_body,
        out_type=jax.ShapeDtypeStruct((NC, N_NODES, AW), jnp.float32),
        mesh=mesh,
        scratch_types=[
            pltpu.VMEM_SHARED((N_NODES, AW), jnp.float32),
        ],
        compiler_params=pltpu.CompilerParams(use_tc_tiling_on_sc=False),
    )
    partials = edge_k(z, row, col)

    out = pl.pallas_call(
        _combine_body,
        grid=(grid,),
        in_specs=[
            pl.BlockSpec((NC, blk, AW), lambda i: (0, i, 0)),
            pl.BlockSpec((HEADS, FEAT), lambda i: (0, 0)),
        ],
        out_specs=pl.BlockSpec((blk, FEAT), lambda i: (i, 0)),
        out_shape=jax.ShapeDtypeStruct((N_NODES, FEAT), jnp.float32),
    )(partials, jnp.asarray(_RMAT))
    return out


# speedup vs baseline: 1.2701x; 1.2701x over previous
import jax
import jax.numpy as jnp
import numpy as np
from jax import lax
from jax.experimental import pallas as pl
from jax.experimental.pallas import tpu as pltpu
from jax.experimental.pallas import tpu_sc as plsc

N_NODES = 10000
IN_CH = 128
OUT_CH = 16
HEADS = 8
FEAT = HEADS * OUT_CH
ZW = 160
PW = 80
AW = 144

NC = 2
NS = 16
NW = NC * NS

EDGE_K = 100
CHUNKS = 100
PHASE = 50
DRAIN = (100, 100, 100, 100, 100, 100, 25)

_JREP8 = np.repeat(np.eye(HEADS, dtype=np.float32), OUT_CH, axis=0)
_JFULL = np.zeros((FEAT, ZW), np.float32)
_JFULL[:, :FEAT] = np.repeat(_JREP8, OUT_CH, axis=1)
_JFULL[:, FEAT:FEAT + HEADS] = _JREP8
_RMAT = np.repeat(np.eye(HEADS, dtype=np.float32), OUT_CH, axis=1)


def _proj_body(x_ref, w_ref, af_ref, j_ref, z_ref):
    xp = jnp.dot(x_ref[...], w_ref[...], preferred_element_type=jnp.float32)
    m = xp * af_ref[...]
    e = jnp.exp(jnp.dot(m, j_ref[...], preferred_element_type=jnp.float32))
    b = jnp.concatenate(
        [xp, jnp.ones((xp.shape[0], ZW - FEAT), jnp.float32)], axis=1)
    z = b * e
    lo = z[:, :PW].astype(jnp.bfloat16).astype(jnp.float32)
    hi = z[:, PW:].astype(jnp.bfloat16).astype(jnp.float32)
    lo_b = jax.lax.bitcast_convert_type(lo, jnp.int32)
    hi_b = jax.lax.bitcast_convert_type(hi, jnp.int32)
    z_ref[...] = jax.lax.shift_right_logical(lo_b, 16) | (
        hi_b & jnp.int32(-65536))


def _combine_body(p_ref, r_ref, o_ref):
    t = p_ref[0] + p_ref[1]
    u = t[:, :FEAT]
    s = t[:, FEAT:FEAT + HEADS]
    srep = jnp.dot(s, r_ref[...], preferred_element_type=jnp.float32)
    o_ref[...] = u / (srep + 1e-16)


def _edge_body(z_hbm, row_hbm, col_hbm, out_hbm, accum_ref):
    cid = lax.axis_index("c")
    sid = lax.axis_index("s")
    wid = cid * NS + sid

    def scoped(row_v, col_v, bb_a, bb_b, fbuf, gs_a, gs_b, ss):
        def zrow(i, _):
            for o in range(AW // 16):
                fbuf[i, pl.ds(o * 16, 16)] = jnp.zeros((16,), jnp.float32)
            return 0
        lax.fori_loop(0, EDGE_K, zrow, 0)
        rows_per_tile = N_NODES // NS
        r0 = sid * rows_per_tile
        for n in DRAIN:
            pltpu.sync_copy(fbuf.at[pl.ds(0, n)], accum_ref.at[pl.ds(r0, n)])
            r0 += n
        plsc.subcore_barrier()

        def convert(bb):
            def crow(r2, _):
                for u in range(2):
                    r = 2 * r2 + u
                    for g in range(5):
                        w = bb[r, pl.ds(g * 16, 16)]
                        lo = jax.lax.bitcast_convert_type(
                            jnp.left_shift(w, 16), jnp.float32)
                        fbuf[r, pl.ds(g * 16, 16)] = lo
                        if g < 4:
                            hi = jax.lax.bitcast_convert_type(
                                jnp.bitwise_and(w, jnp.int32(-65536)), jnp.float32)
                            fbuf[r, pl.ds(PW + g * 16, 16)] = hi
                return 0
            lax.fori_loop(0, EDGE_K // 2, crow, 0)

        def phase(ph, _):
            base = wid * CHUNKS + ph * PHASE
            pltpu.sync_copy(row_hbm.at[pl.ds(base, PHASE)], row_v)
            pltpu.sync_copy(col_hbm.at[pl.ds(base, PHASE)], col_v)
            pltpu.async_copy(z_hbm.at[row_v.at[0]], bb_a, gs_a)

            def group(g, _):
                j0 = 2 * g
                pltpu.make_async_copy(z_hbm.at[row_v.at[j0]], bb_a, gs_a).wait()
                pltpu.async_copy(z_hbm.at[row_v.at[j0 + 1]], bb_b, gs_b)

                @pl.when(g > 0)
                def _():
                    pltpu.make_async_copy(
                        fbuf, accum_ref.at[col_v.at[j0 - 1]], ss).wait()

                convert(bb_a)
                pltpu.async_copy(fbuf, accum_ref.at[col_v.at[j0]], ss, add=True)
                pltpu.make_async_copy(z_hbm.at[row_v.at[j0 + 1]], bb_b, gs_b).wait()

                @pl.when(g < PHASE // 2 - 1)
                def _():
                    pltpu.async_copy(z_hbm.at[row_v.at[j0 + 2]], bb_a, gs_a)

                pltpu.make_async_copy(
                    fbuf, accum_ref.at[col_v.at[j0]], ss).wait()
                convert(bb_b)
                pltpu.async_copy(fbuf, accum_ref.at[col_v.at[j0 + 1]], ss, add=True)
                return 0
            lax.fori_loop(0, PHASE // 2, group, 0)
            pltpu.make_async_copy(
                fbuf, accum_ref.at[col_v.at[PHASE - 1]], ss).wait()
            return 0
        lax.fori_loop(0, CHUNKS // PHASE, phase, 0)
        plsc.subcore_barrier()

        r0 = sid * rows_per_tile
        for n in DRAIN:
            pltpu.sync_copy(accum_ref.at[pl.ds(r0, n)], fbuf.at[pl.ds(0, n)])
            pltpu.sync_copy(fbuf.at[pl.ds(0, n)], out_hbm.at[cid, pl.ds(r0, n)])
            r0 += n

    pl.run_scoped(
        scoped,
        pltpu.VMEM((PHASE, EDGE_K), jnp.int32),
        pltpu.VMEM((PHASE, EDGE_K), jnp.int32),
        pltpu.VMEM((EDGE_K, PW), jnp.int32),
        pltpu.VMEM((EDGE_K, PW), jnp.int32),
        pltpu.VMEM((EDGE_K, AW), jnp.float32),
        pltpu.SemaphoreType.DMA,
        pltpu.SemaphoreType.DMA,
        pltpu.SemaphoreType.DMA,
    )


def kernel(x, edge_index, W, a_src, a_dst):
    row = edge_index[0].astype(jnp.int32).reshape(-1, EDGE_K)
    col = edge_index[1].astype(jnp.int32).reshape(-1, EDGE_K)
    a_flat = a_src.reshape(1, FEAT)

    blk = 1000
    grid = N_NODES // blk
    z = pl.pallas_call(
        _proj_body,
        grid=(grid,),
        in_specs=[
            pl.BlockSpec((blk, IN_CH), lambda i: (i, 0)),
            pl.BlockSpec((IN_CH, FEAT), lambda i: (0, 0)),
            pl.BlockSpec((1, FEAT), lambda i: (0, 0)),
            pl.BlockSpec((IN_CH, ZW), lambda i: (0, 0)),
        ],
        out_specs=pl.BlockSpec((blk, PW), lambda i: (i, 0)),
        out_shape=jax.ShapeDtypeStruct((N_NODES, PW), jnp.int32),
    )(x, W, a_flat, jnp.asarray(_JFULL))

    mesh = plsc.VectorSubcoreMesh(
        core_axis_name="c", subcore_axis_name="s", num_cores=NC, num_subcores=NS)
    edge_k = pl.kernel(
        _edge_body,
        out_type=jax.ShapeDtypeStruct((NC, N_NODES, AW), jnp.float32),
        mesh=mesh,
        scratch_types=[
            pltpu.VMEM_SHARED((N_NODES, AW), jnp.float32),
        ],
        compiler_params=pltpu.CompilerParams(use_tc_tiling_on_sc=False),
    )
    partials = edge_k(z, row, col)

    out = pl.pallas_call(
        _combine_body,
        grid=(grid,),
        in_specs=[
            pl.BlockSpec((NC, blk, AW), lambda i: (0, i, 0)),
            pl.BlockSpec((HEADS, FEAT), lambda i: (0, 0)),
        ],
        out_specs=pl.BlockSpec((blk, FEAT), lambda i: (i, 0)),
        out_shape=jax.ShapeDtypeStruct((N_NODES, FEAT), jnp.float32),
    )(partials, jnp.asarray(_RMAT))
    return out
